# Initial kernel scaffold; baseline (speedup 1.0000x reference)
import jax
import jax.numpy as jnp
from jax.experimental import pallas as pl


def kernel(z, codebooks):
    raise NotImplementedError("write your pallas kernel here")


# speedup vs baseline: 2.5020x; 2.5020x over previous
import jax
import jax.numpy as jnp
from jax.experimental import pallas as pl

_B, _C, _T = 16, 256, 2048
_K, _S = 1024, 4
_ROWS = _B * _T
_TILE = 2048
_GRID = _ROWS // _TILE


def _rvq_body(z_ref, cb_ref, cbt_ref, cnorm_ref, zq_ref, idx_ref, loss_ref):
    r = z_ref[...]
    zq = jnp.zeros_like(r)
    iota_k = jax.lax.broadcasted_iota(jnp.int32, (_TILE, _K), 1)
    lane = jax.lax.broadcasted_iota(jnp.int32, (1, 128), 1)
    loss_vec = jnp.zeros((1, 128), jnp.float32)
    for s in range(_S):
        scores = jax.lax.dot_general(
            r, cbt_ref[s], (((1,), (0,)), ((), ())),
            preferred_element_type=jnp.float32)
        rnorm = jnp.sum(r * r, axis=1, keepdims=True)
        dist = (rnorm + cnorm_ref[s]) - 2.0 * scores
        m = jnp.min(dist, axis=1, keepdims=True)
        idx = jnp.min(jnp.where(dist == m, iota_k, _K), axis=1)
        idx_ref[0, s, :] = idx
        onehot = (iota_k == idx[:, None]).astype(jnp.float32)
        q = jax.lax.dot_general(
            onehot, cb_ref[s], (((1,), (0,)), ((), ())),
            preferred_element_type=jnp.float32)
        zq = zq + q
        r = r - q
        loss_vec = loss_vec + jnp.where(lane == s, jnp.sum(r * r), 0.0)
    zq_ref[...] = zq
    loss_ref[0] = loss_vec


def kernel(z, codebooks):
    z_flat = jnp.transpose(z, (0, 2, 1)).reshape(_ROWS, _C)
    cbt = jnp.transpose(codebooks, (0, 2, 1))
    cnorm = jnp.sum(codebooks ** 2, axis=2)[:, None, :]
    zq_flat, indices, losses = pl.pallas_call(
        _rvq_body,
        grid=(_GRID,),
        in_specs=[
            pl.BlockSpec((_TILE, _C), lambda b: (b, 0)),
            pl.BlockSpec((_S, _K, _C), lambda b: (0, 0, 0)),
            pl.BlockSpec((_S, _C, _K), lambda b: (0, 0, 0)),
            pl.BlockSpec((_S, 1, _K), lambda b: (0, 0, 0)),
        ],
        out_specs=[
            pl.BlockSpec((_TILE, _C), lambda b: (b, 0)),
            pl.BlockSpec((1, _S, _T), lambda b: (b, 0, 0)),
            pl.BlockSpec((1, 1, 128), lambda b: (b, 0, 0)),
        ],
        out_shape=[
            jax.ShapeDtypeStruct((_ROWS, _C), jnp.float32),
            jax.ShapeDtypeStruct((_GRID, _S, _T), jnp.int32),
            jax.ShapeDtypeStruct((_GRID, 1, 128), jnp.float32),
        ],
    )(z_flat, codebooks, cbt, cnorm)
    zq = jnp.transpose(zq_flat.reshape(_B, _T, _C), (0, 2, 1))
    loss = jnp.sum(losses[:, 0, :_S]) / (_B * _C * _T) / _S
    return zq, indices, loss


# speedup vs baseline: 3.7744x; 1.5086x over previous
import jax
import jax.numpy as jnp
from jax.experimental import pallas as pl

_B, _C, _T = 16, 256, 2048
_K, _S = 1024, 4


def _rvq_body(z_ref, cb_ref, cbt_ref, zq_ref, idx_ref, loss_ref):
    r = z_ref[0]
    zq = jnp.zeros_like(r)
    iota_k = jax.lax.broadcasted_iota(jnp.int32, (_K, _T), 0)
    lane = jax.lax.broadcasted_iota(jnp.int32, (1, 128), 1)
    loss_vec = jnp.zeros((1, 128), jnp.float32)
    for s in range(_S):
        cb = cb_ref[s]
        scores = jax.lax.dot_general(
            cb, r, (((1,), (0,)), ((), ())),
            preferred_element_type=jnp.float32)
        rnorm = jnp.sum(r * r, axis=0, keepdims=True)
        if s > 0:
            loss_vec = loss_vec + jnp.where(lane == s - 1, jnp.sum(rnorm), 0.0)
        cnorm = jnp.sum(cb * cb, axis=1, keepdims=True)
        dist = (rnorm + cnorm) - 2.0 * scores
        m = jnp.min(dist, axis=0, keepdims=True)
        idx = jnp.min(jnp.where(dist == m, iota_k, _K), axis=0)
        idx_ref[0, s, :] = idx
        onehot = (iota_k == idx[None, :]).astype(jnp.float32)
        q = jax.lax.dot_general(
            cbt_ref[s], onehot, (((1,), (0,)), ((), ())),
            preferred_element_type=jnp.float32)
        zq = zq + q
        r = r - q
    loss_vec = loss_vec + jnp.where(lane == _S - 1, jnp.sum(r * r), 0.0)
    zq_ref[0] = zq
    loss_ref[0] = loss_vec


def kernel(z, codebooks):
    cbt = jnp.transpose(codebooks, (0, 2, 1))
    zq, indices, losses = pl.pallas_call(
        _rvq_body,
        grid=(_B,),
        in_specs=[
            pl.BlockSpec((1, _C, _T), lambda b: (b, 0, 0)),
            pl.BlockSpec((_S, _K, _C), lambda b: (0, 0, 0)),
            pl.BlockSpec((_S, _C, _K), lambda b: (0, 0, 0)),
        ],
        out_specs=[
            pl.BlockSpec((1, _C, _T), lambda b: (b, 0, 0)),
            pl.BlockSpec((1, _S, _T), lambda b: (b, 0, 0)),
            pl.BlockSpec((1, 1, 128), lambda b: (b, 0, 0)),
        ],
        out_shape=[
            jax.ShapeDtypeStruct((_B, _C, _T), jnp.float32),
            jax.ShapeDtypeStruct((_B, _S, _T), jnp.int32),
            jax.ShapeDtypeStruct((_B, 1, 128), jnp.float32),
        ],
    )(z, codebooks, cbt)
    loss = jnp.sum(losses[:, 0, :_S]) / (_B * _C * _T) / _S
    return zq, indices, loss


# speedup vs baseline: 3.9338x; 1.0422x over previous
import jax
import jax.numpy as jnp
from jax.experimental import pallas as pl

_B, _C, _T = 16, 256, 2048
_K, _S = 1024, 4


def _rvq_body(z_ref, cb2_ref, cbt_ref, zq_ref, idx_ref, loss_ref):
    r = z_ref[0]
    zq = jnp.zeros_like(r)
    iota_k = jax.lax.broadcasted_iota(jnp.int32, (_K, _T), 0)
    lane = jax.lax.broadcasted_iota(jnp.int32, (1, 128), 1)
    loss_vec = jnp.zeros((1, 128), jnp.float32)
    for s in range(_S):
        cb2 = cb2_ref[s]
        scores2 = jax.lax.dot_general(
            cb2, r, (((1,), (0,)), ((), ())),
            preferred_element_type=jnp.float32)
        rnorm = jnp.sum(r * r, axis=0, keepdims=True)
        if s > 0:
            loss_vec = loss_vec + jnp.where(lane == s - 1, jnp.sum(rnorm), 0.0)
        cnorm = jnp.sum(cb2 * cb2, axis=1, keepdims=True) * 0.25
        dist = (rnorm + cnorm) - scores2
        m = jnp.min(dist, axis=0, keepdims=True)
        idx = jnp.min(jnp.where(dist == m, iota_k, _K), axis=0)
        idx_ref[0, s, :] = idx
        onehot = (iota_k == idx[None, :]).astype(jnp.float32)
        q = jax.lax.dot_general(
            cbt_ref[s], onehot, (((1,), (0,)), ((), ())),
            preferred_element_type=jnp.float32)
        zq = zq + q
        r = r - q
    loss_vec = loss_vec + jnp.where(lane == _S - 1, jnp.sum(r * r), 0.0)
    zq_ref[0] = zq
    loss_ref[0] = loss_vec


def kernel(z, codebooks):
    cb2 = codebooks * 2.0
    cbt = jnp.transpose(codebooks, (0, 2, 1))
    zq, indices, losses = pl.pallas_call(
        _rvq_body,
        grid=(_B,),
        in_specs=[
            pl.BlockSpec((1, _C, _T), lambda b: (b, 0, 0)),
            pl.BlockSpec((_S, _K, _C), lambda b: (0, 0, 0)),
            pl.BlockSpec((_S, _C, _K), lambda b: (0, 0, 0)),
        ],
        out_specs=[
            pl.BlockSpec((1, _C, _T), lambda b: (b, 0, 0)),
            pl.BlockSpec((1, _S, _T), lambda b: (b, 0, 0)),
            pl.BlockSpec((1, 1, 128), lambda b: (b, 0, 0)),
        ],
        out_shape=[
            jax.ShapeDtypeStruct((_B, _C, _T), jnp.float32),
            jax.ShapeDtypeStruct((_B, _S, _T), jnp.int32),
            jax.ShapeDtypeStruct((_B, 1, 128), jnp.float32),
        ],
    )(z, cb2, cbt)
    loss = jnp.sum(losses[:, 0, :_S]) / (_B * _C * _T) / _S
    return zq, indices, loss
